# baseline (speedup 1.0000x reference)
import functools

import jax
import jax.numpy as jnp
from jax import lax
from jax.experimental import pallas as pl
from jax.experimental.pallas import tpu as pltpu

_N_BOXES = 20000
_PRE_TOPK = 2000
_POST_TOPK = 1000
_NMS_THRESH = 0.7
_NPAD = 32768
_ROWS = 256
_LANES = 128
_TOP_ROWS = 16
_NEG_INF = float("-inf")


def _bitonic_stage(arrs, ks_pos, ki_pos, d, blk, ri, ci, descending):
  if d < _LANES:
    def partner(a):
      lo = pltpu.roll(a, _LANES - d, axis=1)
      hi = pltpu.roll(a, d, axis=1)
      return jnp.where((ci & d) != 0, hi, lo)
  else:
    m = d // _LANES
    rows = arrs[0].shape[0]
    g = rows // (2 * m)
    def partner(a):
      a4 = a.reshape(g, 2, m, _LANES)
      a4 = jnp.concatenate([a4[:, 1:2], a4[:, 0:1]], axis=1)
      return a4.reshape(rows, _LANES)

  parts = [partner(a) for a in arrs]
  p = ri * _LANES + ci
  key_s, sq = arrs[ks_pos], parts[ks_pos]
  if descending:
    key_i, iq = arrs[ki_pos], parts[ki_pos]
    mine_first = (key_s > sq) | ((key_s == sq) & (key_i < iq))
  else:
    mine_first = key_s < sq
  am_high = (p & d) != 0
  blk_flip = (p & blk) != 0
  keep_mine = (mine_first != am_high) != blk_flip
  return [jnp.where(keep_mine, a, q) for a, q in zip(arrs, parts)]


def _bitonic_sort(arrs, ks_pos, ki_pos, n, ri, ci, descending):
  blk = 2
  while blk <= n:
    d = blk // 2
    while d >= 1:
      arrs = _bitonic_stage(arrs, ks_pos, ki_pos, d, blk, ri, ci,
                            descending)
      d //= 2
    blk *= 2
  return arrs


def _bitonic_stage_cm(arrs, ks_pos, ki_pos, d, blk, p, ri, ci, rows,
                      descending):
  if d < rows:
    m = d
    g = rows // (2 * m)
    def partner(a):
      a4 = a.reshape(g, 2, m, _LANES)
      a4 = jnp.concatenate([a4[:, 1:2], a4[:, 0:1]], axis=1)
      return a4.reshape(rows, _LANES)
  else:
    dl = d // rows
    def partner(a):
      lo = pltpu.roll(a, _LANES - dl, axis=1)
      hi = pltpu.roll(a, dl, axis=1)
      return jnp.where((ci & dl) != 0, hi, lo)

  parts = [partner(a) for a in arrs]
  key_s, sq = arrs[ks_pos], parts[ks_pos]
  if descending:
    key_i, iq = arrs[ki_pos], parts[ki_pos]
    mine_first = (key_s > sq) | ((key_s == sq) & (key_i < iq))
  else:
    mine_first = key_s < sq
  am_high = (p & d) != 0
  blk_flip = (p & blk) != 0
  keep_mine = (mine_first != am_high) != blk_flip
  return [jnp.where(keep_mine, a, q) for a, q in zip(arrs, parts)]


def _bitonic_sort_cm(arrs, ks_pos, ki_pos, n, ri, ci, rows, descending):
  p = ci * rows + ri
  blk = 2
  while blk <= n:
    d = blk // 2
    while d >= 1:
      arrs = _bitonic_stage_cm(arrs, ks_pos, ki_pos, d, blk, p, ri, ci,
                               rows, descending)
      d //= 2
    blk *= 2
  return arrs


def _extract_top_cm(a, lanes):
  t = jnp.transpose(a[:, :lanes])
  lo = t[:, :_LANES]
  hi = t[:, _LANES:]
  return jnp.stack([lo, hi], axis=1).reshape(2 * lanes, _LANES)


def _transpose(x, eye):
  del eye
  return jnp.transpose(x)


def _nms_kernel(s_ref, x1_ref, y1_ref, x2_ref, y2_ref,
                os_ref, ox1_ref, oy1_ref, ox2_ref, oy2_ref, m_ref):
  ri = lax.broadcasted_iota(jnp.int32, (_ROWS, _LANES), 0)
  ci = lax.broadcasted_iota(jnp.int32, (_ROWS, _LANES), 1)

  s = s_ref[...]
  idx = ri * _LANES + ci
  arrs = _bitonic_sort_cm([s, idx], 0, 1, _NPAD, ri, ci, _ROWS,
                          descending=True)
  s, idxs = arrs

  st = _extract_top_cm(s, 8)
  idxt = _extract_top_cm(idxs, 8)
  ri16 = ri[:_TOP_ROWS]
  ci16 = ci[:_TOP_ROWS]
  rank = ri16 * _LANES + ci16

  def split3(c):
    hi = c.astype(jnp.bfloat16)
    r1 = c - hi.astype(jnp.float32)
    lo = r1.astype(jnp.bfloat16)
    llo = (r1 - lo.astype(jnp.float32)).astype(jnp.bfloat16)
    return hi, lo, llo

  coords = [x1_ref[...], y1_ref[...], x2_ref[...], y2_ref[...]]
  splits = [split3(c) for c in coords]
  cat_hi = jnp.concatenate([sp[0] for sp in splits], axis=1)
  cat_lo = jnp.concatenate([sp[1] for sp in splits], axis=1)
  cat_llo = jnp.concatenate([sp[2] for sp in splits], axis=1)

  io2_r = lax.broadcasted_iota(jnp.int32, (2 * _LANES, _ROWS), 1)
  io_r = lax.broadcasted_iota(jnp.int32, (_LANES, _LANES), 0)
  io_c = lax.broadcasted_iota(jnp.int32, (_LANES, _LANES), 1)

  idxcols = []
  for g in range(_TOP_ROWS):
    idxcols.append(jnp.transpose(idxt[g:g + 1, :]))

  cx1, cy1, cx2, cy2 = [], [], [], []
  for g in range(0, _TOP_ROWS, 2):
    two_col = jnp.concatenate([idxcols[g], idxcols[g + 1]], axis=0)
    a_oh = (two_col // _LANES == io2_r).astype(jnp.bfloat16)
    prod = (
        jnp.dot(a_oh, cat_hi, preferred_element_type=jnp.float32) +
        jnp.dot(a_oh, cat_lo, preferred_element_type=jnp.float32) +
        jnp.dot(a_oh, cat_llo, preferred_element_type=jnp.float32))
    lane_sel = two_col % _LANES
    for half in range(2):
      bh = (lane_sel[half * _LANES:(half + 1) * _LANES] ==
            io_c).astype(jnp.float32)
      pr = prod[half * _LANES:(half + 1) * _LANES]
      for k, dst in enumerate((cx1, cy1, cx2, cy2)):
        sel = pr[:, k * _LANES:(k + 1) * _LANES] * bh
        dst.append(jnp.sum(sel, axis=1, keepdims=True))

  carea = [(cx2[g] - cx1[g]) * (cy2[g] - cy1[g])
           for g in range(_TOP_ROWS)]
  rx1 = [jnp.transpose(c) for c in cx1]
  ry1 = [jnp.transpose(c) for c in cy1]
  rx2 = [jnp.transpose(c) for c in cx2]
  ry2 = [jnp.transpose(c) for c in cy2]
  rarea = [jnp.transpose(c) for c in carea]

  x1t = jnp.concatenate(rx1, axis=0)
  y1t = jnp.concatenate(ry1, axis=0)
  x2t = jnp.concatenate(rx2, axis=0)
  y2t = jnp.concatenate(ry2, axis=0)

  w = x2t - x1t
  h = y2t - y1t
  valid = (rank < _PRE_TOPK) & (w >= 0.0) & (h >= 0.0)
  validf = valid.astype(jnp.float32)

  eye = None

  tile_of = {}
  t = 0
  for b in range(_TOP_ROWS):
    for a in range(b + 1):
      tile_of[(a, b)] = t
      t += 1
  for b in range(_TOP_ROWS):
    xb1 = rx1[b]
    yb1 = ry1[b]
    xb2 = rx2[b]
    yb2 = ry2[b]
    ab = rarea[b]
    for a in range(b + 1):
      iw = jnp.clip(jnp.minimum(cx2[a], xb2) - jnp.maximum(cx1[a], xb1),
                    0.0)
      ih = jnp.clip(jnp.minimum(cy2[a], yb2) - jnp.maximum(cy1[a], yb1),
                    0.0)
      inter = iw * ih
      union = carea[a] + ab - inter
      over = inter / jnp.maximum(union, 1e-9) > _NMS_THRESH
      if a == b:
        over = over & (io_r < io_c)
      ofs = tile_of[(a, b)] * _LANES
      m_ref[ofs:ofs + _LANES, :] = over.astype(jnp.float32)

  def col(row_vec):
    return jnp.transpose(row_vec)

  def sweep(k):
    cols_old = _transpose(k, eye)
    new_rows = []
    for b in range(_TOP_ROWS):
      acc = jnp.zeros((1, _LANES), jnp.float32)
      for a in range(b + 1):
        ofs = tile_of[(a, b)] * _LANES
        acc = acc + jnp.sum(m_ref[ofs:ofs + _LANES, :] * cols_old[:, a:a + 1],
                            axis=0, keepdims=True)
      row = validf[b:b + 1, :] * (acc <= 0.0).astype(jnp.float32)
      new_rows.append(row)
    return jnp.concatenate(new_rows, axis=0)

  k1 = sweep(validf)
  k2 = sweep(k1)
  k3 = sweep(k2)

  def wbody(carry):
    k, _ = carry
    ka = sweep(k)
    kb = sweep(ka)
    return kb, jnp.sum(jnp.abs(kb - ka)) == 0.0

  kfin, _ = lax.while_loop(
      lambda c: jnp.logical_not(c[1]), wbody,
      (k3, jnp.sum(jnp.abs(k3 - k2)) == 0.0))

  kept = kfin > 0.0
  out_s = jnp.where(kept, st, _NEG_INF)
  key = rank + jnp.where(kept, 0, 4096)
  arrs2 = [key, out_s, x1t, y1t, x2t, y2t]
  arrs2 = _bitonic_sort(arrs2, 0, None, _TOP_ROWS * _LANES, ri16, ci16,
                        descending=False)
  _, fs, fx1, fy1, fx2, fy2 = arrs2

  os_ref[...] = fs[:8]
  ox1_ref[...] = fx1[:8]
  oy1_ref[...] = fy1[:8]
  ox2_ref[...] = fx2[:8]
  oy2_ref[...] = fy2[:8]


@jax.jit
def kernel(boxes, scores):
  spad = jnp.full((_NPAD,), _NEG_INF, jnp.float32).at[:_N_BOXES].set(scores)
  coords = []
  for c in range(4):
    coords.append(
        jnp.zeros((_NPAD,), jnp.float32).at[:_N_BOXES].set(boxes[:, c])
        .reshape(_ROWS, _LANES))
  s2d = spad.reshape(_ROWS, _LANES)

  out_shapes = [jax.ShapeDtypeStruct((8, _LANES), jnp.float32)] * 5
  outs = pl.pallas_call(
      _nms_kernel,
      out_shape=out_shapes,
      scratch_shapes=[pltpu.VMEM((136 * _LANES, _LANES), jnp.float32)],
  )(s2d, *coords)
  fs, fx1, fy1, fx2, fy2 = outs
  out_s = fs.reshape(8 * _LANES)[:_POST_TOPK]
  out_b = jnp.stack(
      [fx1.reshape(8 * _LANES)[:_POST_TOPK],
       fy1.reshape(8 * _LANES)[:_POST_TOPK],
       fx2.reshape(8 * _LANES)[:_POST_TOPK],
       fy2.reshape(8 * _LANES)[:_POST_TOPK]], axis=1)
  return out_b, out_s


# speedup vs baseline: 1.0007x; 1.0007x over previous
import functools

import jax
import jax.numpy as jnp
from jax import lax
from jax.experimental import pallas as pl
from jax.experimental.pallas import tpu as pltpu

_N_BOXES = 20000
_PRE_TOPK = 2000
_POST_TOPK = 1000
_NMS_THRESH = 0.7
_NPAD = 32768
_ROWS = 256
_LANES = 128
_TOP_ROWS = 16
_NEG_INF = float("-inf")


def _bitonic_stage(arrs, ks_pos, ki_pos, d, blk, ri, ci, descending):
  if d < _LANES:
    def partner(a):
      lo = pltpu.roll(a, _LANES - d, axis=1)
      hi = pltpu.roll(a, d, axis=1)
      return jnp.where((ci & d) != 0, hi, lo)
  else:
    m = d // _LANES
    rows = arrs[0].shape[0]
    g = rows // (2 * m)
    def partner(a):
      a4 = a.reshape(g, 2, m, _LANES)
      a4 = jnp.concatenate([a4[:, 1:2], a4[:, 0:1]], axis=1)
      return a4.reshape(rows, _LANES)

  parts = [partner(a) for a in arrs]
  p = ri * _LANES + ci
  key_s, sq = arrs[ks_pos], parts[ks_pos]
  if descending:
    key_i, iq = arrs[ki_pos], parts[ki_pos]
    mine_first = (key_s > sq) | ((key_s == sq) & (key_i < iq))
  else:
    mine_first = key_s < sq
  am_high = (p & d) != 0
  blk_flip = (p & blk) != 0
  keep_mine = (mine_first != am_high) != blk_flip
  return [jnp.where(keep_mine, a, q) for a, q in zip(arrs, parts)]


def _bitonic_sort(arrs, ks_pos, ki_pos, n, ri, ci, descending):
  blk = 2
  while blk <= n:
    d = blk // 2
    while d >= 1:
      arrs = _bitonic_stage(arrs, ks_pos, ki_pos, d, blk, ri, ci,
                            descending)
      d //= 2
    blk *= 2
  return arrs


def _bitonic_stage_cm(arrs, ks_pos, ki_pos, d, blk, p, ri, ci, rows,
                      descending):
  if d < rows:
    m = d
    g = rows // (2 * m)
    def partner(a):
      a4 = a.reshape(g, 2, m, _LANES)
      a4 = jnp.concatenate([a4[:, 1:2], a4[:, 0:1]], axis=1)
      return a4.reshape(rows, _LANES)
  else:
    dl = d // rows
    def partner(a):
      lo = pltpu.roll(a, _LANES - dl, axis=1)
      hi = pltpu.roll(a, dl, axis=1)
      return jnp.where((ci & dl) != 0, hi, lo)

  parts = [partner(a) for a in arrs]
  key_s, sq = arrs[ks_pos], parts[ks_pos]
  if descending:
    key_i, iq = arrs[ki_pos], parts[ki_pos]
    mine_first = (key_s > sq) | ((key_s == sq) & (key_i < iq))
  else:
    mine_first = key_s < sq
  am_high = (p & d) != 0
  blk_flip = (p & blk) != 0
  keep_mine = (mine_first != am_high) != blk_flip
  return [jnp.where(keep_mine, a, q) for a, q in zip(arrs, parts)]


def _bitonic_sort_cm(arrs, ks_pos, ki_pos, n, ri, ci, rows, descending):
  p = ci * rows + ri
  blk = 2
  while blk <= n:
    d = blk // 2
    while d >= 1:
      arrs = _bitonic_stage_cm(arrs, ks_pos, ki_pos, d, blk, p, ri, ci,
                               rows, descending)
      d //= 2
    blk *= 2
  return arrs


def _extract_top_cm(a, lanes):
  t = jnp.transpose(a[:, :lanes])
  lo = t[:, :_LANES]
  hi = t[:, _LANES:]
  return jnp.stack([lo, hi], axis=1).reshape(2 * lanes, _LANES)


def _transpose(x, eye):
  del eye
  return jnp.transpose(x)


def _nms_kernel(s_ref, x1_ref, y1_ref, x2_ref, y2_ref,
                os_ref, ox1_ref, oy1_ref, ox2_ref, oy2_ref, m_ref):
  ri = lax.broadcasted_iota(jnp.int32, (_ROWS, _LANES), 0)
  ci = lax.broadcasted_iota(jnp.int32, (_ROWS, _LANES), 1)

  s = s_ref[...]
  idx = ri * _LANES + ci
  arrs = _bitonic_sort_cm([s, idx], 0, 1, _NPAD, ri, ci, _ROWS,
                          descending=True)
  s, idxs = arrs

  st = _extract_top_cm(s, 8)
  idxt = _extract_top_cm(idxs, 8)
  ri16 = ri[:_TOP_ROWS]
  ci16 = ci[:_TOP_ROWS]
  rank = ri16 * _LANES + ci16

  def split3(c):
    hi = c.astype(jnp.bfloat16)
    r1 = c - hi.astype(jnp.float32)
    lo = r1.astype(jnp.bfloat16)
    llo = (r1 - lo.astype(jnp.float32)).astype(jnp.bfloat16)
    return hi, lo, llo

  coords = [x1_ref[...], y1_ref[...], x2_ref[...], y2_ref[...]]
  splits = [split3(c) for c in coords]
  cat_hi = jnp.concatenate([sp[0] for sp in splits], axis=1)
  cat_lo = jnp.concatenate([sp[1] for sp in splits], axis=1)
  cat_llo = jnp.concatenate([sp[2] for sp in splits], axis=1)

  io2_r = lax.broadcasted_iota(jnp.int32, (2 * _LANES, _ROWS), 1)
  io_r = lax.broadcasted_iota(jnp.int32, (_LANES, _LANES), 0)
  io_c = lax.broadcasted_iota(jnp.int32, (_LANES, _LANES), 1)

  idxcols = []
  for g in range(_TOP_ROWS):
    idxcols.append(jnp.transpose(idxt[g:g + 1, :]))

  cx1, cy1, cx2, cy2 = [], [], [], []
  for g in range(0, _TOP_ROWS, 2):
    two_col = jnp.concatenate([idxcols[g], idxcols[g + 1]], axis=0)
    a_oh = (two_col // _LANES == io2_r).astype(jnp.bfloat16)
    prod = (
        jnp.dot(a_oh, cat_hi, preferred_element_type=jnp.float32) +
        jnp.dot(a_oh, cat_lo, preferred_element_type=jnp.float32) +
        jnp.dot(a_oh, cat_llo, preferred_element_type=jnp.float32))
    lane_sel = two_col % _LANES
    for half in range(2):
      bh = (lane_sel[half * _LANES:(half + 1) * _LANES] ==
            io_c).astype(jnp.float32)
      pr = prod[half * _LANES:(half + 1) * _LANES]
      for k, dst in enumerate((cx1, cy1, cx2, cy2)):
        sel = pr[:, k * _LANES:(k + 1) * _LANES] * bh
        dst.append(jnp.sum(sel, axis=1, keepdims=True))

  carea = [(cx2[g] - cx1[g]) * (cy2[g] - cy1[g])
           for g in range(_TOP_ROWS)]
  rx1 = [jnp.transpose(c) for c in cx1]
  ry1 = [jnp.transpose(c) for c in cy1]
  rx2 = [jnp.transpose(c) for c in cx2]
  ry2 = [jnp.transpose(c) for c in cy2]
  rarea = [jnp.transpose(c) for c in carea]

  x1t = jnp.concatenate(rx1, axis=0)
  y1t = jnp.concatenate(ry1, axis=0)
  x2t = jnp.concatenate(rx2, axis=0)
  y2t = jnp.concatenate(ry2, axis=0)

  w = x2t - x1t
  h = y2t - y1t
  valid = (rank < _PRE_TOPK) & (w >= 0.0) & (h >= 0.0)
  validf = valid.astype(jnp.float32)

  eye = None

  tile_of = {}
  t = 0
  for b in range(_TOP_ROWS):
    for a in range(b + 1):
      tile_of[(a, b)] = t
      t += 1
  for b in range(_TOP_ROWS):
    xb1 = rx1[b]
    yb1 = ry1[b]
    xb2 = rx2[b]
    yb2 = ry2[b]
    ab = rarea[b]
    for a in range(b + 1):
      iw = jnp.clip(jnp.minimum(cx2[a], xb2) - jnp.maximum(cx1[a], xb1),
                    0.0)
      ih = jnp.clip(jnp.minimum(cy2[a], yb2) - jnp.maximum(cy1[a], yb1),
                    0.0)
      inter = iw * ih
      union = carea[a] + ab - inter
      over = inter / jnp.maximum(union, 1e-9) > _NMS_THRESH
      if a == b:
        over = over & (io_r < io_c)
      ofs = tile_of[(a, b)] * _LANES
      m_ref[ofs:ofs + _LANES, :] = over.astype(jnp.float32)

  def col(row_vec):
    return jnp.transpose(row_vec)

  def sweep(k):
    cols_old = _transpose(k, eye)
    new_rows = []
    for b in range(_TOP_ROWS):
      acc = jnp.zeros((1, _LANES), jnp.float32)
      for a in range(b + 1):
        ofs = tile_of[(a, b)] * _LANES
        acc = acc + jnp.sum(m_ref[ofs:ofs + _LANES, :] * cols_old[:, a:a + 1],
                            axis=0, keepdims=True)
      row = validf[b:b + 1, :] * (acc <= 0.0).astype(jnp.float32)
      new_rows.append(row)
    return jnp.concatenate(new_rows, axis=0)

  k1 = sweep(validf)
  k2 = sweep(k1)

  def wbody(carry):
    k, _ = carry
    ka = sweep(k)
    return ka, jnp.sum(jnp.abs(ka - k)) == 0.0

  kfin, _ = lax.while_loop(
      lambda c: jnp.logical_not(c[1]), wbody,
      (k2, jnp.sum(jnp.abs(k2 - k1)) == 0.0))

  kept = kfin > 0.0
  out_s = jnp.where(kept, st, _NEG_INF)
  key = rank + jnp.where(kept, 0, 4096)
  arrs2 = [key, out_s, x1t, y1t, x2t, y2t]
  arrs2 = _bitonic_sort(arrs2, 0, None, _TOP_ROWS * _LANES, ri16, ci16,
                        descending=False)
  _, fs, fx1, fy1, fx2, fy2 = arrs2

  os_ref[...] = fs[:8]
  ox1_ref[...] = fx1[:8]
  oy1_ref[...] = fy1[:8]
  ox2_ref[...] = fx2[:8]
  oy2_ref[...] = fy2[:8]


@jax.jit
def kernel(boxes, scores):
  spad = jnp.full((_NPAD,), _NEG_INF, jnp.float32).at[:_N_BOXES].set(scores)
  coords = []
  for c in range(4):
    coords.append(
        jnp.zeros((_NPAD,), jnp.float32).at[:_N_BOXES].set(boxes[:, c])
        .reshape(_ROWS, _LANES))
  s2d = spad.reshape(_ROWS, _LANES)

  out_shapes = [jax.ShapeDtypeStruct((8, _LANES), jnp.float32)] * 5
  outs = pl.pallas_call(
      _nms_kernel,
      out_shape=out_shapes,
      scratch_shapes=[pltpu.VMEM((136 * _LANES, _LANES), jnp.float32)],
  )(s2d, *coords)
  fs, fx1, fy1, fx2, fy2 = outs
  out_s = fs.reshape(8 * _LANES)[:_POST_TOPK]
  out_b = jnp.stack(
      [fx1.reshape(8 * _LANES)[:_POST_TOPK],
       fy1.reshape(8 * _LANES)[:_POST_TOPK],
       fx2.reshape(8 * _LANES)[:_POST_TOPK],
       fy2.reshape(8 * _LANES)[:_POST_TOPK]], axis=1)
  return out_b, out_s


# speedup vs baseline: 1.0377x; 1.0370x over previous
import functools

import jax
import jax.numpy as jnp
from jax import lax
from jax.experimental import pallas as pl
from jax.experimental.pallas import tpu as pltpu

_N_BOXES = 20000
_PRE_TOPK = 2000
_POST_TOPK = 1000
_NMS_THRESH = 0.7
_NPAD = 32768
_ROWS = 256
_LANES = 128
_TOP_ROWS = 16
_NEG_INF = float("-inf")


def _bitonic_stage(arrs, ks_pos, ki_pos, d, blk, ri, ci, descending):
  if d < _LANES:
    def partner(a):
      lo = pltpu.roll(a, _LANES - d, axis=1)
      hi = pltpu.roll(a, d, axis=1)
      return jnp.where((ci & d) != 0, hi, lo)
  else:
    m = d // _LANES
    rows = arrs[0].shape[0]
    g = rows // (2 * m)
    def partner(a):
      a4 = a.reshape(g, 2, m, _LANES)
      a4 = jnp.concatenate([a4[:, 1:2], a4[:, 0:1]], axis=1)
      return a4.reshape(rows, _LANES)

  parts = [partner(a) for a in arrs]
  p = ri * _LANES + ci
  key_s, sq = arrs[ks_pos], parts[ks_pos]
  if descending:
    key_i, iq = arrs[ki_pos], parts[ki_pos]
    mine_first = (key_s > sq) | ((key_s == sq) & (key_i < iq))
  else:
    mine_first = key_s < sq
  am_high = (p & d) != 0
  blk_flip = (p & blk) != 0
  keep_mine = (mine_first != am_high) != blk_flip
  return [jnp.where(keep_mine, a, q) for a, q in zip(arrs, parts)]


def _bitonic_sort(arrs, ks_pos, ki_pos, n, ri, ci, descending):
  blk = 2
  while blk <= n:
    d = blk // 2
    while d >= 1:
      arrs = _bitonic_stage(arrs, ks_pos, ki_pos, d, blk, ri, ci,
                            descending)
      d //= 2
    blk *= 2
  return arrs


def _bitonic_stage_cm(arrs, ks_pos, ki_pos, d, blk, p, ri, ci, rows,
                      descending):
  if d < rows:
    m = d
    g = rows // (2 * m)
    def partner(a):
      a4 = a.reshape(g, 2, m, _LANES)
      a4 = jnp.concatenate([a4[:, 1:2], a4[:, 0:1]], axis=1)
      return a4.reshape(rows, _LANES)
  else:
    dl = d // rows
    def partner(a):
      lo = pltpu.roll(a, _LANES - dl, axis=1)
      hi = pltpu.roll(a, dl, axis=1)
      return jnp.where((ci & dl) != 0, hi, lo)

  parts = [partner(a) for a in arrs]
  key_s, sq = arrs[ks_pos], parts[ks_pos]
  if descending:
    key_i, iq = arrs[ki_pos], parts[ki_pos]
    mine_first = (key_s > sq) | ((key_s == sq) & (key_i < iq))
  else:
    mine_first = key_s < sq
  am_high = (p & d) != 0
  blk_flip = (p & blk) != 0
  keep_mine = (mine_first != am_high) != blk_flip
  return [jnp.where(keep_mine, a, q) for a, q in zip(arrs, parts)]


def _bitonic_sort_cm(arrs, ks_pos, ki_pos, n, ri, ci, rows, descending):
  p = ci * rows + ri
  blk = 2
  while blk <= n:
    d = blk // 2
    while d >= 1:
      arrs = _bitonic_stage_cm(arrs, ks_pos, ki_pos, d, blk, p, ri, ci,
                               rows, descending)
      d //= 2
    blk *= 2
  return arrs


def _extract_top_cm(a, lanes):
  t = jnp.transpose(a[:, :lanes])
  lo = t[:, :_LANES]
  hi = t[:, _LANES:]
  return jnp.stack([lo, hi], axis=1).reshape(2 * lanes, _LANES)


def _transpose(x, eye):
  del eye
  return jnp.transpose(x)


def _nms_kernel(s_ref, x1_ref, y1_ref, x2_ref, y2_ref,
                os_ref, ox1_ref, oy1_ref, ox2_ref, oy2_ref, m_ref):
  ri = lax.broadcasted_iota(jnp.int32, (_ROWS, _LANES), 0)
  ci = lax.broadcasted_iota(jnp.int32, (_ROWS, _LANES), 1)

  s = s_ref[...]
  idx = ri * _LANES + ci
  arrs = _bitonic_sort_cm([s, idx], 0, 1, _NPAD, ri, ci, _ROWS,
                          descending=True)
  s, idxs = arrs

  st = _extract_top_cm(s, 8)
  idxt = _extract_top_cm(idxs, 8)
  ri16 = ri[:_TOP_ROWS]
  ci16 = ci[:_TOP_ROWS]
  rank = ri16 * _LANES + ci16

  def split3(c):
    hi = c.astype(jnp.bfloat16)
    r1 = c - hi.astype(jnp.float32)
    lo = r1.astype(jnp.bfloat16)
    llo = (r1 - lo.astype(jnp.float32)).astype(jnp.bfloat16)
    return hi, lo, llo

  coords = [x1_ref[...], y1_ref[...], x2_ref[...], y2_ref[...]]
  splits = [split3(c) for c in coords]
  cat_hi = jnp.concatenate([sp[0] for sp in splits], axis=1)
  cat_lo = jnp.concatenate([sp[1] for sp in splits], axis=1)
  cat_llo = jnp.concatenate([sp[2] for sp in splits], axis=1)

  io2_r = lax.broadcasted_iota(jnp.int32, (2 * _LANES, _ROWS), 1)
  io_r = lax.broadcasted_iota(jnp.int32, (_LANES, _LANES), 0)
  io_c = lax.broadcasted_iota(jnp.int32, (_LANES, _LANES), 1)

  idxcols = []
  for g in range(_TOP_ROWS):
    idxcols.append(jnp.transpose(idxt[g:g + 1, :]))

  cx1, cy1, cx2, cy2 = [], [], [], []
  for g in range(0, _TOP_ROWS, 2):
    two_col = jnp.concatenate([idxcols[g], idxcols[g + 1]], axis=0)
    a_oh = (two_col // _LANES == io2_r).astype(jnp.bfloat16)
    prod = (
        jnp.dot(a_oh, cat_hi, preferred_element_type=jnp.float32) +
        jnp.dot(a_oh, cat_lo, preferred_element_type=jnp.float32) +
        jnp.dot(a_oh, cat_llo, preferred_element_type=jnp.float32))
    lane_sel = two_col % _LANES
    for half in range(2):
      bh = (lane_sel[half * _LANES:(half + 1) * _LANES] ==
            io_c).astype(jnp.float32)
      pr = prod[half * _LANES:(half + 1) * _LANES]
      for k, dst in enumerate((cx1, cy1, cx2, cy2)):
        sel = pr[:, k * _LANES:(k + 1) * _LANES] * bh
        dst.append(jnp.sum(sel, axis=1, keepdims=True))

  carea = [(cx2[g] - cx1[g]) * (cy2[g] - cy1[g])
           for g in range(_TOP_ROWS)]
  rx1 = [jnp.transpose(c) for c in cx1]
  ry1 = [jnp.transpose(c) for c in cy1]
  rx2 = [jnp.transpose(c) for c in cx2]
  ry2 = [jnp.transpose(c) for c in cy2]
  rarea = [jnp.transpose(c) for c in carea]

  x1t = jnp.concatenate(rx1, axis=0)
  y1t = jnp.concatenate(ry1, axis=0)
  x2t = jnp.concatenate(rx2, axis=0)
  y2t = jnp.concatenate(ry2, axis=0)

  w = x2t - x1t
  h = y2t - y1t
  valid = (rank < _PRE_TOPK) & (w >= 0.0) & (h >= 0.0)
  validf = valid.astype(jnp.float32)

  eye = None

  tile_of = {}
  t = 0
  for b in range(_TOP_ROWS):
    for a in range(b + 1):
      tile_of[(a, b)] = t
      t += 1
  for b in range(_TOP_ROWS):
    xb1 = rx1[b]
    yb1 = ry1[b]
    xb2 = rx2[b]
    yb2 = ry2[b]
    ab = rarea[b]
    for a in range(b + 1):
      iw = jnp.clip(jnp.minimum(cx2[a], xb2) - jnp.maximum(cx1[a], xb1),
                    0.0)
      ih = jnp.clip(jnp.minimum(cy2[a], yb2) - jnp.maximum(cy1[a], yb1),
                    0.0)
      inter = iw * ih
      union = carea[a] + ab - inter
      over = inter / jnp.maximum(union, 1e-9) > _NMS_THRESH
      if a == b:
        over = over & (io_r < io_c)
      ofs = tile_of[(a, b)] * _LANES
      m_ref[ofs:ofs + _LANES, :] = over.astype(jnp.float32)

  def col(row_vec):
    return jnp.transpose(row_vec)

  def sweep(carry):
    k, _ = carry
    cols_old = _transpose(k, eye)
    new_rows = []
    for b in range(_TOP_ROWS):
      acc = jnp.zeros((1, _LANES), jnp.float32)
      for a in range(b + 1):
        ofs = tile_of[(a, b)] * _LANES
        acc = acc + jnp.sum(m_ref[ofs:ofs + _LANES, :] * cols_old[:, a:a + 1],
                            axis=0, keepdims=True)
      row = validf[b:b + 1, :] * (acc <= 0.0).astype(jnp.float32)
      new_rows.append(row)
    knew = jnp.concatenate(new_rows, axis=0)
    done = jnp.sum(jnp.abs(knew - k)) == 0.0
    return knew, done

  k0 = validf
  kfin, _ = lax.while_loop(lambda c: jnp.logical_not(c[1]), sweep,
                           (k0, jnp.asarray(False)))

  kept = kfin > 0.0
  out_s = jnp.where(kept, st, _NEG_INF)
  key = rank + jnp.where(kept, 0, 4096)
  arrs2 = [key, out_s, x1t, y1t, x2t, y2t]
  arrs2 = _bitonic_sort(arrs2, 0, None, _TOP_ROWS * _LANES, ri16, ci16,
                        descending=False)
  _, fs, fx1, fy1, fx2, fy2 = arrs2

  os_ref[...] = fs[:8]
  ox1_ref[...] = fx1[:8]
  oy1_ref[...] = fy1[:8]
  ox2_ref[...] = fx2[:8]
  oy2_ref[...] = fy2[:8]


@jax.jit
def kernel(boxes, scores):
  spad = jnp.full((_NPAD,), _NEG_INF, jnp.float32).at[:_N_BOXES].set(scores)
  coords = []
  for c in range(4):
    coords.append(
        jnp.zeros((_NPAD,), jnp.float32).at[:_N_BOXES].set(boxes[:, c])
        .reshape(_ROWS, _LANES))
  s2d = spad.reshape(_ROWS, _LANES)

  out_shapes = [jax.ShapeDtypeStruct((8, _LANES), jnp.float32)] * 5
  outs = pl.pallas_call(
      _nms_kernel,
      out_shape=out_shapes,
      scratch_shapes=[pltpu.VMEM((136 * _LANES, _LANES), jnp.float32)],
  )(s2d, *coords)
  fs, fx1, fy1, fx2, fy2 = outs
  out_s = fs.reshape(8 * _LANES)[:_POST_TOPK]
  out_b = jnp.stack(
      [fx1.reshape(8 * _LANES)[:_POST_TOPK],
       fy1.reshape(8 * _LANES)[:_POST_TOPK],
       fx2.reshape(8 * _LANES)[:_POST_TOPK],
       fy2.reshape(8 * _LANES)[:_POST_TOPK]], axis=1)
  return out_b, out_s
